# Initial kernel scaffold; baseline (speedup 1.0000x reference)
import jax
import jax.numpy as jnp
from jax.experimental import pallas as pl


def kernel(x, pos, support_points, neighbors_indices, W_fc1, W_fc2, W_fc3, g1, b1, g2, b2, alpha, beta, norm_radius, W_cv):
    raise NotImplementedError("write your pallas kernel here")


# speedup vs baseline: 1.1297x; 1.1297x over previous
import functools

import jax
import jax.numpy as jnp
from jax import lax
from jax.experimental import pallas as pl
from jax.experimental.pallas import tpu as pltpu
from jax.experimental.pallas import tpu_sc as plsc

F32 = jnp.float32

_N = 10000
_S = 10000
_K = 16
_C = 128
_O = 128
_M = 16
_SK = _S * _K
_EPS = 1e-5

_NC, _NS = 2, 16
_NW = _NC * _NS
_PER_W = _SK // _NW
_CH = 128
_NFULL = _PER_W // _CH
_TAIL = _PER_W - _NFULL * _CH

_TS1 = 1000
_TS3 = 400


def _sc_gather_build():
    mesh = plsc.VectorSubcoreMesh(core_axis_name="c", subcore_axis_name="s")

    @functools.partial(
        pl.kernel,
        mesh=mesh,
        out_type=[
            jax.ShapeDtypeStruct((_SK, _C), F32),
            jax.ShapeDtypeStruct((_SK, 16), F32),
        ],
        scratch_types=[
            pltpu.VMEM((_CH,), jnp.int32),
            pltpu.VMEM((_CH, _C), F32),
            pltpu.VMEM((_CH, 16), F32),
            pltpu.VMEM((_TAIL,), jnp.int32),
            pltpu.VMEM((_TAIL, _C), F32),
            pltpu.VMEM((_TAIL, 16), F32),
            pltpu.SemaphoreType.DMA,
            pltpu.SemaphoreType.DMA,
        ],
        compiler_params=pltpu.CompilerParams(use_tc_tiling_on_sc=False),
    )
    def k(xt_hbm, pt_hbm, idx_hbm, xg_hbm, pg_hbm,
          idx_v, xrow_v, prow_v, idxt_v, xrowt_v, prowt_v, sem1, sem2):
        wid = lax.axis_index("s") * _NC + lax.axis_index("c")
        base = wid * _PER_W

        def body(i, carry):
            off = base + i * _CH
            pltpu.sync_copy(idx_hbm.at[pl.ds(off, _CH)], idx_v)
            cp1 = pltpu.async_copy(xt_hbm.at[idx_v], xrow_v, sem1)
            cp2 = pltpu.async_copy(pt_hbm.at[idx_v], prow_v, sem2)
            cp1.wait()
            cp2.wait()
            pltpu.sync_copy(xrow_v, xg_hbm.at[pl.ds(off, _CH)])
            pltpu.sync_copy(prow_v, pg_hbm.at[pl.ds(off, _CH)])
            return carry

        lax.fori_loop(0, _NFULL, body, 0)

        off = base + _NFULL * _CH
        pltpu.sync_copy(idx_hbm.at[pl.ds(off, _TAIL)], idxt_v)
        cp1 = pltpu.async_copy(xt_hbm.at[idxt_v], xrowt_v, sem1)
        cp2 = pltpu.async_copy(pt_hbm.at[idxt_v], prowt_v, sem2)
        cp1.wait()
        cp2.wait()
        pltpu.sync_copy(xrowt_v, xg_hbm.at[pl.ds(off, _TAIL)])
        pltpu.sync_copy(prowt_v, pg_hbm.at[pl.ds(off, _TAIL)])

    return k


def _affine_from_stats(sum_v, sumsq_v, g, b):
    mu = sum_v / _SK
    var = sumsq_v / _SK - mu * mu
    scale = g * lax.rsqrt(var + _EPS)
    shift = b - mu * scale
    return scale, shift


def _pos_stage(pg, sp, w1r, scale1, shift1, a, bb, inv_r, ts):
    tsk = ts * _K
    sp_rep = jnp.broadcast_to(sp.reshape(ts, 1, 16), (ts, _K, 16)).reshape(tsk, 16)
    ptsr = pg - sp_rep
    d2 = jnp.sum(ptsr * ptsr, axis=1, keepdims=True)
    d = jnp.sqrt(d2)
    dw = jax.nn.sigmoid(-a * d + bb)
    dws = jnp.sum(dw.reshape(ts, _K), axis=1, keepdims=True)
    dws = dws + (dws == 0).astype(F32) + 1e-6
    dwn = (dw.reshape(ts, _K) / dws * float(_K)).reshape(tsk, 1)
    ptsn = ptsr * inv_r
    m1p = jnp.dot(ptsn, w1r, preferred_element_type=F32)
    mat1 = jnp.maximum(m1p * scale1 + shift1, 0.0)
    wm1 = mat1 * dwn
    mp1 = jnp.max(wm1.reshape(ts, _K, 16), axis=1, keepdims=True)
    mp1r = jnp.broadcast_to(mp1, (ts, _K, 16)).reshape(tsk, 16)
    return dwn, m1p, mat1, mp1r


def _pass1_body(pg_ref, sp_ref, w1r_ref, scl_ref, s1_ref, q1_ref):
    inv_r = 1.0 / scl_ref[0, 2]
    sp = sp_ref[...]
    pg = pg_ref[...]
    ts = sp.shape[0]
    tsk = ts * _K
    sp_rep = jnp.broadcast_to(sp.reshape(ts, 1, 16), (ts, _K, 16)).reshape(tsk, 16)
    ptsn = (pg - sp_rep) * inv_r
    m1p = jnp.dot(ptsn, w1r_ref[...], preferred_element_type=F32)
    ps = jnp.sum(m1p, axis=0, keepdims=True)
    pq = jnp.sum(m1p * m1p, axis=0, keepdims=True)

    @pl.when(pl.program_id(0) == 0)
    def _():
        s1_ref[...] = jnp.zeros_like(s1_ref)
        q1_ref[...] = jnp.zeros_like(q1_ref)

    s1_ref[...] += ps
    q1_ref[...] += pq


def _pass2_body(pg_ref, sp_ref, w1r_ref, w2a_ref, w2b_ref, scl_ref,
                g1_ref, b1_ref, s1_ref, q1_ref, s2_ref, q2_ref):
    a = scl_ref[0, 0]
    bb = scl_ref[0, 1]
    inv_r = 1.0 / scl_ref[0, 2]
    scale1, shift1 = _affine_from_stats(s1_ref[...], q1_ref[...],
                                        g1_ref[...], b1_ref[...])
    ts = sp_ref.shape[0]
    _, _, mat1, mp1r = _pos_stage(pg_ref[...], sp_ref[...], w1r_ref[...],
                                  scale1, shift1, a, bb, inv_r, ts)
    m2p = (jnp.dot(mat1, w2a_ref[...], preferred_element_type=F32)
           + jnp.dot(mp1r, w2b_ref[...], preferred_element_type=F32))
    ps = jnp.sum(m2p, axis=0, keepdims=True)
    pq = jnp.sum(m2p * m2p, axis=0, keepdims=True)

    @pl.when(pl.program_id(0) == 0)
    def _():
        s2_ref[...] = jnp.zeros_like(s2_ref)
        q2_ref[...] = jnp.zeros_like(q2_ref)

    s2_ref[...] += ps
    q2_ref[...] += pq


def _pass3_body(pg_ref, sp_ref, xg_ref, w1r_ref, w2a_ref, w2b_ref,
                w3a_ref, w3b_ref, wcv_ref, scl_ref,
                g1_ref, b1_ref, g2_ref, b2_ref,
                s1_ref, q1_ref, s2_ref, q2_ref, out_ref):
    a = scl_ref[0, 0]
    bb = scl_ref[0, 1]
    inv_r = 1.0 / scl_ref[0, 2]
    scale1, shift1 = _affine_from_stats(s1_ref[...], q1_ref[...],
                                        g1_ref[...], b1_ref[...])
    scale2, shift2 = _affine_from_stats(s2_ref[...], q2_ref[...],
                                        g2_ref[...], b2_ref[...])
    ts = sp_ref.shape[0]
    tsk = ts * _K
    dwn, _, mat1, mp1r = _pos_stage(pg_ref[...], sp_ref[...], w1r_ref[...],
                                    scale1, shift1, a, bb, inv_r, ts)
    m2p = (jnp.dot(mat1, w2a_ref[...], preferred_element_type=F32)
           + jnp.dot(mp1r, w2b_ref[...], preferred_element_type=F32))
    mat2 = jnp.maximum(m2p * scale2 + shift2, 0.0)
    wm2 = mat2 * dwn
    mp2 = jnp.max(wm2.reshape(ts, _K, 16), axis=1, keepdims=True)
    mp2r = jnp.broadcast_to(mp2, (ts, _K, 16)).reshape(tsk, 16)
    m3p = (jnp.dot(mat2, w3a_ref[...], preferred_element_type=F32)
           + jnp.dot(mp2r, w3b_ref[...], preferred_element_type=F32))
    mat = jnp.maximum(m3p, 0.0) * dwn

    xg = xg_ref[...]
    acc = jnp.zeros((ts, _O), F32)
    for m in range(_M):
        xw = xg * mat[:, m:m + 1]
        fm = jnp.sum(xw.reshape(ts, _K, _C), axis=1)
        acc = acc + jnp.dot(fm, wcv_ref[m * _C:(m + 1) * _C, :],
                            preferred_element_type=F32)
    out_ref[...] = acc


def _full16(_): return (0, 0)


def _make_pass1(interpret=False):
    grid = (_S // _TS1,)
    return pl.pallas_call(
        _pass1_body,
        grid=grid,
        in_specs=[
            pl.BlockSpec((_TS1 * _K, 16), lambda i: (i, 0)),
            pl.BlockSpec((_TS1, 16), lambda i: (i, 0)),
            pl.BlockSpec((16, 16), _full16),
            pl.BlockSpec((1, 4), _full16),
        ],
        out_specs=[pl.BlockSpec((1, 16), _full16),
                   pl.BlockSpec((1, 16), _full16)],
        out_shape=[jax.ShapeDtypeStruct((1, 16), F32),
                   jax.ShapeDtypeStruct((1, 16), F32)],
        interpret=interpret,
    )


def _make_pass2(interpret=False):
    grid = (_S // _TS1,)
    return pl.pallas_call(
        _pass2_body,
        grid=grid,
        in_specs=[
            pl.BlockSpec((_TS1 * _K, 16), lambda i: (i, 0)),
            pl.BlockSpec((_TS1, 16), lambda i: (i, 0)),
            pl.BlockSpec((16, 16), _full16),
            pl.BlockSpec((16, 16), _full16),
            pl.BlockSpec((16, 16), _full16),
            pl.BlockSpec((1, 4), _full16),
            pl.BlockSpec((1, 16), _full16),
            pl.BlockSpec((1, 16), _full16),
            pl.BlockSpec((1, 16), _full16),
            pl.BlockSpec((1, 16), _full16),
        ],
        out_specs=[pl.BlockSpec((1, 16), _full16),
                   pl.BlockSpec((1, 16), _full16)],
        out_shape=[jax.ShapeDtypeStruct((1, 16), F32),
                   jax.ShapeDtypeStruct((1, 16), F32)],
        interpret=interpret,
    )


def _make_pass3(interpret=False):
    grid = (_S // _TS3,)
    return pl.pallas_call(
        _pass3_body,
        grid=grid,
        in_specs=[
            pl.BlockSpec((_TS3 * _K, 16), lambda i: (i, 0)),
            pl.BlockSpec((_TS3, 16), lambda i: (i, 0)),
            pl.BlockSpec((_TS3 * _K, _C), lambda i: (i, 0)),
            pl.BlockSpec((16, 16), _full16),
            pl.BlockSpec((16, 16), _full16),
            pl.BlockSpec((16, 16), _full16),
            pl.BlockSpec((16, 16), _full16),
            pl.BlockSpec((16, 16), _full16),
            pl.BlockSpec((_M * _C, _O), _full16),
            pl.BlockSpec((1, 4), _full16),
            pl.BlockSpec((1, 16), _full16),
            pl.BlockSpec((1, 16), _full16),
            pl.BlockSpec((1, 16), _full16),
            pl.BlockSpec((1, 16), _full16),
            pl.BlockSpec((1, 16), _full16),
            pl.BlockSpec((1, 16), _full16),
            pl.BlockSpec((1, 16), _full16),
            pl.BlockSpec((1, 16), _full16),
        ],
        out_specs=pl.BlockSpec((_TS3, _O), lambda i: (i, 0)),
        out_shape=jax.ShapeDtypeStruct((_S, _O), F32),
        interpret=interpret,
    )


def _prep_weights(W_fc1, W_fc2, W_fc3, W_cv):
    w1r = jnp.pad(W_fc1, ((0, 0), (0, 13))).T
    w2a = W_fc2[:, :16].T
    w2b = W_fc2[:, 16:].T
    w3a = W_fc3[:, :16].T
    w3b = W_fc3[:, 16:].T
    wcv = jnp.transpose(W_cv, (2, 1, 0)).reshape(_M * _C, _O)
    return w1r, w2a, w2b, w3a, w3b, wcv


def kernel(x, pos, support_points, neighbors_indices, W_fc1, W_fc2, W_fc3,
           g1, b1, g2, b2, alpha, beta, norm_radius, W_cv):
    xt = x[0].T
    pt = jnp.pad(pos[0].T, ((0, 0), (0, 13)))
    spt = jnp.pad(support_points[0].T, ((0, 0), (0, 13)))
    idx = neighbors_indices[0].astype(jnp.int32).reshape(_SK)

    xg, pg = _sc_gather_build()(xt, pt, idx)

    w1r, w2a, w2b, w3a, w3b, wcv = _prep_weights(W_fc1, W_fc2, W_fc3, W_cv)
    scl = jnp.concatenate([alpha, beta, norm_radius,
                           jnp.zeros((1,), F32)]).reshape(1, 4)
    g1r, b1r = g1.reshape(1, 16), b1.reshape(1, 16)
    g2r, b2r = g2.reshape(1, 16), b2.reshape(1, 16)

    s1, q1 = _make_pass1()(pg, spt, w1r, scl)
    s2, q2 = _make_pass2()(pg, spt, w1r, w2a, w2b, scl, g1r, b1r, s1, q1)
    out2d = _make_pass3()(pg, spt, xg, w1r, w2a, w2b, w3a, w3b, wcv, scl,
                          g1r, b1r, g2r, b2r, s1, q1, s2, q2)
    return out2d.T[None, :, :]


# speedup vs baseline: 1.6366x; 1.4487x over previous
import functools

import jax
import jax.numpy as jnp
from jax import lax
from jax.experimental import pallas as pl
from jax.experimental.pallas import tpu as pltpu
from jax.experimental.pallas import tpu_sc as plsc

F32 = jnp.float32

_N = 10000
_S = 10000
_K = 16
_C = 128
_O = 128
_M = 16
_SK = _S * _K
_EPS = 1e-5

_NC, _NS = 2, 16
_NW = _NC * _NS
_PER_W = _SK // _NW
_CH = 128
_NFULL = _PER_W // _CH
_TAIL = _PER_W - _NFULL * _CH

_TS1 = 1000
_TS3 = 400


def _sc_gather_build():
    mesh = plsc.VectorSubcoreMesh(core_axis_name="c", subcore_axis_name="s")

    @functools.partial(
        pl.kernel,
        mesh=mesh,
        out_type=[
            jax.ShapeDtypeStruct((_SK, _C), F32),
            jax.ShapeDtypeStruct((_SK, 16), F32),
        ],
        scratch_types=[
            pltpu.VMEM((_CH,), jnp.int32),
            pltpu.VMEM((_CH, _C), F32),
            pltpu.VMEM((_CH, 16), F32),
            pltpu.VMEM((_TAIL,), jnp.int32),
            pltpu.VMEM((_TAIL, _C), F32),
            pltpu.VMEM((_TAIL, 16), F32),
            pltpu.SemaphoreType.DMA,
            pltpu.SemaphoreType.DMA,
        ],
        compiler_params=pltpu.CompilerParams(use_tc_tiling_on_sc=False),
    )
    def k(xt_hbm, pt_hbm, idx_hbm, xg_hbm, pg_hbm,
          idx_v, xrow_v, prow_v, idxt_v, xrowt_v, prowt_v, sem1, sem2):
        wid = lax.axis_index("s") * _NC + lax.axis_index("c")
        base = wid * _PER_W

        def body(i, carry):
            off = base + i * _CH
            pltpu.sync_copy(idx_hbm.at[pl.ds(off, _CH)], idx_v)
            cp1 = pltpu.async_copy(xt_hbm.at[idx_v], xrow_v, sem1)
            cp2 = pltpu.async_copy(pt_hbm.at[idx_v], prow_v, sem2)
            cp1.wait()
            cp2.wait()
            pltpu.sync_copy(xrow_v, xg_hbm.at[pl.ds(off, _CH)])
            pltpu.sync_copy(prow_v, pg_hbm.at[pl.ds(off, _CH)])
            return carry

        lax.fori_loop(0, _NFULL, body, 0)

        off = base + _NFULL * _CH
        pltpu.sync_copy(idx_hbm.at[pl.ds(off, _TAIL)], idxt_v)
        cp1 = pltpu.async_copy(xt_hbm.at[idxt_v], xrowt_v, sem1)
        cp2 = pltpu.async_copy(pt_hbm.at[idxt_v], prowt_v, sem2)
        cp1.wait()
        cp2.wait()
        pltpu.sync_copy(xrowt_v, xg_hbm.at[pl.ds(off, _TAIL)])
        pltpu.sync_copy(prowt_v, pg_hbm.at[pl.ds(off, _TAIL)])

    return k


def _affine_from_stats(sum_v, sumsq_v, g, b):
    mu = sum_v / _SK
    var = sumsq_v / _SK - mu * mu
    scale = g * lax.rsqrt(var + _EPS)
    shift = b - mu * scale
    return scale, shift


def _pos_stage(pg, sp, w1r, scale1, shift1, a, bb, inv_r, ts):
    tsk = ts * _K
    sp_rep = jnp.broadcast_to(sp.reshape(ts, 1, 16), (ts, _K, 16)).reshape(tsk, 16)
    ptsr = pg - sp_rep
    d2 = jnp.sum(ptsr * ptsr, axis=1, keepdims=True)
    d = jnp.sqrt(d2)
    dw = jax.nn.sigmoid(-a * d + bb)
    dws = jnp.sum(dw.reshape(ts, _K), axis=1, keepdims=True)
    dws = dws + (dws == 0).astype(F32) + 1e-6
    dwn = (dw.reshape(ts, _K) / dws * float(_K)).reshape(tsk, 1)
    ptsn = ptsr * inv_r
    m1p = jnp.dot(ptsn, w1r, preferred_element_type=F32)
    mat1 = jnp.maximum(m1p * scale1 + shift1, 0.0)
    wm1 = mat1 * dwn
    mp1 = jnp.max(wm1.reshape(ts, _K, 16), axis=1, keepdims=True)
    mp1r = jnp.broadcast_to(mp1, (ts, _K, 16)).reshape(tsk, 16)
    return dwn, m1p, mat1, mp1r


def _pass1_body(pg_ref, sp_ref, w1r_ref, scl_ref, s1_ref, q1_ref):
    inv_r = 1.0 / scl_ref[0, 2]
    sp = sp_ref[...]
    pg = pg_ref[...]
    ts = sp.shape[0]
    tsk = ts * _K
    sp_rep = jnp.broadcast_to(sp.reshape(ts, 1, 16), (ts, _K, 16)).reshape(tsk, 16)
    ptsn = (pg - sp_rep) * inv_r
    m1p = jnp.dot(ptsn, w1r_ref[...], preferred_element_type=F32)
    ps = jnp.sum(m1p, axis=0, keepdims=True)
    pq = jnp.sum(m1p * m1p, axis=0, keepdims=True)

    @pl.when(pl.program_id(0) == 0)
    def _():
        s1_ref[...] = jnp.zeros_like(s1_ref)
        q1_ref[...] = jnp.zeros_like(q1_ref)

    s1_ref[...] += ps
    q1_ref[...] += pq


def _pass2_body(pg_ref, sp_ref, w1r_ref, w2a_ref, w2b_ref, scl_ref,
                g1_ref, b1_ref, s1_ref, q1_ref, s2_ref, q2_ref):
    a = scl_ref[0, 0]
    bb = scl_ref[0, 1]
    inv_r = 1.0 / scl_ref[0, 2]
    scale1, shift1 = _affine_from_stats(s1_ref[...], q1_ref[...],
                                        g1_ref[...], b1_ref[...])
    ts = sp_ref.shape[0]
    _, _, mat1, mp1r = _pos_stage(pg_ref[...], sp_ref[...], w1r_ref[...],
                                  scale1, shift1, a, bb, inv_r, ts)
    m2p = (jnp.dot(mat1, w2a_ref[...], preferred_element_type=F32)
           + jnp.dot(mp1r, w2b_ref[...], preferred_element_type=F32))
    ps = jnp.sum(m2p, axis=0, keepdims=True)
    pq = jnp.sum(m2p * m2p, axis=0, keepdims=True)

    @pl.when(pl.program_id(0) == 0)
    def _():
        s2_ref[...] = jnp.zeros_like(s2_ref)
        q2_ref[...] = jnp.zeros_like(q2_ref)

    s2_ref[...] += ps
    q2_ref[...] += pq


def _pass3_body(pg_ref, sp_ref, xg_ref, w1r_ref, w2a_ref, w2b_ref,
                w3a_ref, w3b_ref, wcv_ref, scl_ref,
                g1_ref, b1_ref, g2_ref, b2_ref,
                s1_ref, q1_ref, s2_ref, q2_ref, out_ref):
    a = scl_ref[0, 0]
    bb = scl_ref[0, 1]
    inv_r = 1.0 / scl_ref[0, 2]
    scale1, shift1 = _affine_from_stats(s1_ref[...], q1_ref[...],
                                        g1_ref[...], b1_ref[...])
    scale2, shift2 = _affine_from_stats(s2_ref[...], q2_ref[...],
                                        g2_ref[...], b2_ref[...])
    ts = sp_ref.shape[0]
    tsk = ts * _K
    dwn, _, mat1, mp1r = _pos_stage(pg_ref[...], sp_ref[...], w1r_ref[...],
                                    scale1, shift1, a, bb, inv_r, ts)
    m2p = (jnp.dot(mat1, w2a_ref[...], preferred_element_type=F32)
           + jnp.dot(mp1r, w2b_ref[...], preferred_element_type=F32))
    mat2 = jnp.maximum(m2p * scale2 + shift2, 0.0)
    wm2 = mat2 * dwn
    mp2 = jnp.max(wm2.reshape(ts, _K, 16), axis=1, keepdims=True)
    mp2r = jnp.broadcast_to(mp2, (ts, _K, 16)).reshape(tsk, 16)
    m3p = (jnp.dot(mat2, w3a_ref[...], preferred_element_type=F32)
           + jnp.dot(mp2r, w3b_ref[...], preferred_element_type=F32))
    mat = jnp.maximum(m3p, 0.0) * dwn

    xg = xg_ref[...]
    feats = lax.dot_general(
        mat.reshape(ts, _K, _M), xg.reshape(ts, _K, _C),
        dimension_numbers=(((1,), (1,)), ((0,), (0,))),
        preferred_element_type=F32)
    acc = jnp.zeros((ts, _O), F32)
    for m in range(_M):
        acc = acc + jnp.dot(feats[:, m, :], wcv_ref[m * _C:(m + 1) * _C, :],
                            preferred_element_type=F32)
    out_ref[...] = acc


def _full16(_): return (0, 0)


def _make_pass1(interpret=False):
    grid = (_S // _TS1,)
    return pl.pallas_call(
        _pass1_body,
        grid=grid,
        in_specs=[
            pl.BlockSpec((_TS1 * _K, 16), lambda i: (i, 0)),
            pl.BlockSpec((_TS1, 16), lambda i: (i, 0)),
            pl.BlockSpec((16, 16), _full16),
            pl.BlockSpec((1, 4), _full16),
        ],
        out_specs=[pl.BlockSpec((1, 16), _full16),
                   pl.BlockSpec((1, 16), _full16)],
        out_shape=[jax.ShapeDtypeStruct((1, 16), F32),
                   jax.ShapeDtypeStruct((1, 16), F32)],
        interpret=interpret,
    )


def _make_pass2(interpret=False):
    grid = (_S // _TS1,)
    return pl.pallas_call(
        _pass2_body,
        grid=grid,
        in_specs=[
            pl.BlockSpec((_TS1 * _K, 16), lambda i: (i, 0)),
            pl.BlockSpec((_TS1, 16), lambda i: (i, 0)),
            pl.BlockSpec((16, 16), _full16),
            pl.BlockSpec((16, 16), _full16),
            pl.BlockSpec((16, 16), _full16),
            pl.BlockSpec((1, 4), _full16),
            pl.BlockSpec((1, 16), _full16),
            pl.BlockSpec((1, 16), _full16),
            pl.BlockSpec((1, 16), _full16),
            pl.BlockSpec((1, 16), _full16),
        ],
        out_specs=[pl.BlockSpec((1, 16), _full16),
                   pl.BlockSpec((1, 16), _full16)],
        out_shape=[jax.ShapeDtypeStruct((1, 16), F32),
                   jax.ShapeDtypeStruct((1, 16), F32)],
        interpret=interpret,
    )


def _make_pass3(interpret=False):
    grid = (_S // _TS3,)
    return pl.pallas_call(
        _pass3_body,
        grid=grid,
        in_specs=[
            pl.BlockSpec((_TS3 * _K, 16), lambda i: (i, 0)),
            pl.BlockSpec((_TS3, 16), lambda i: (i, 0)),
            pl.BlockSpec((_TS3 * _K, _C), lambda i: (i, 0)),
            pl.BlockSpec((16, 16), _full16),
            pl.BlockSpec((16, 16), _full16),
            pl.BlockSpec((16, 16), _full16),
            pl.BlockSpec((16, 16), _full16),
            pl.BlockSpec((16, 16), _full16),
            pl.BlockSpec((_M * _C, _O), _full16),
            pl.BlockSpec((1, 4), _full16),
            pl.BlockSpec((1, 16), _full16),
            pl.BlockSpec((1, 16), _full16),
            pl.BlockSpec((1, 16), _full16),
            pl.BlockSpec((1, 16), _full16),
            pl.BlockSpec((1, 16), _full16),
            pl.BlockSpec((1, 16), _full16),
            pl.BlockSpec((1, 16), _full16),
            pl.BlockSpec((1, 16), _full16),
        ],
        out_specs=pl.BlockSpec((_TS3, _O), lambda i: (i, 0)),
        out_shape=jax.ShapeDtypeStruct((_S, _O), F32),
        interpret=interpret,
    )


def _prep_weights(W_fc1, W_fc2, W_fc3, W_cv):
    w1r = jnp.pad(W_fc1, ((0, 0), (0, 13))).T
    w2a = W_fc2[:, :16].T
    w2b = W_fc2[:, 16:].T
    w3a = W_fc3[:, :16].T
    w3b = W_fc3[:, 16:].T
    wcv = jnp.transpose(W_cv, (2, 1, 0)).reshape(_M * _C, _O)
    return w1r, w2a, w2b, w3a, w3b, wcv


def kernel(x, pos, support_points, neighbors_indices, W_fc1, W_fc2, W_fc3,
           g1, b1, g2, b2, alpha, beta, norm_radius, W_cv):
    xt = x[0].T
    pt = jnp.pad(pos[0].T, ((0, 0), (0, 13)))
    spt = jnp.pad(support_points[0].T, ((0, 0), (0, 13)))
    idx = neighbors_indices[0].astype(jnp.int32).reshape(_SK)

    xg, pg = _sc_gather_build()(xt, pt, idx)

    w1r, w2a, w2b, w3a, w3b, wcv = _prep_weights(W_fc1, W_fc2, W_fc3, W_cv)
    scl = jnp.concatenate([alpha, beta, norm_radius,
                           jnp.zeros((1,), F32)]).reshape(1, 4)
    g1r, b1r = g1.reshape(1, 16), b1.reshape(1, 16)
    g2r, b2r = g2.reshape(1, 16), b2.reshape(1, 16)

    s1, q1 = _make_pass1()(pg, spt, w1r, scl)
    s2, q2 = _make_pass2()(pg, spt, w1r, w2a, w2b, scl, g1r, b1r, s1, q1)
    out2d = _make_pass3()(pg, spt, xg, w1r, w2a, w2b, w3a, w3b, wcv, scl,
                          g1r, b1r, g2r, b2r, s1, q1, s2, q2)
    return out2d.T[None, :, :]


# speedup vs baseline: 1.7445x; 1.0660x over previous
import functools

import jax
import jax.numpy as jnp
from jax import lax
from jax.experimental import pallas as pl
from jax.experimental.pallas import tpu as pltpu
from jax.experimental.pallas import tpu_sc as plsc

F32 = jnp.float32

_N = 10000
_S = 10000
_K = 16
_C = 128
_O = 128
_M = 16
_SK = _S * _K
_EPS = 1e-5

_NC, _NS = 2, 16
_NW = _NC * _NS
_PER_W = _SK // _NW
_CH = 128
_NFULL = _PER_W // _CH
_TAIL = _PER_W - _NFULL * _CH

_TS1 = 1000
_TS3 = 400


def _sc_gather_build(width):
    mesh = plsc.VectorSubcoreMesh(core_axis_name="c", subcore_axis_name="s")

    @functools.partial(
        pl.kernel,
        mesh=mesh,
        out_type=jax.ShapeDtypeStruct((_SK, width), F32),
        scratch_types=[
            pltpu.VMEM((_CH,), jnp.int32),
            pltpu.VMEM((_CH, width), F32),
            pltpu.VMEM((_TAIL,), jnp.int32),
            pltpu.VMEM((_TAIL, width), F32),
            pltpu.SemaphoreType.DMA,
        ],
        compiler_params=pltpu.CompilerParams(use_tc_tiling_on_sc=False),
    )
    def k(tab_hbm, idx_hbm, out_hbm, idx_v, row_v, idxt_v, rowt_v, sem):
        wid = lax.axis_index("s") * _NC + lax.axis_index("c")
        base = wid * _PER_W

        def body(i, carry):
            off = base + i * _CH
            pltpu.sync_copy(idx_hbm.at[pl.ds(off, _CH)], idx_v)
            pltpu.async_copy(tab_hbm.at[idx_v], row_v, sem).wait()
            pltpu.sync_copy(row_v, out_hbm.at[pl.ds(off, _CH)])
            return carry

        lax.fori_loop(0, _NFULL, body, 0)

        off = base + _NFULL * _CH
        pltpu.sync_copy(idx_hbm.at[pl.ds(off, _TAIL)], idxt_v)
        pltpu.async_copy(tab_hbm.at[idxt_v], rowt_v, sem).wait()
        pltpu.sync_copy(rowt_v, out_hbm.at[pl.ds(off, _TAIL)])

    return k


def _affine_from_stats(sum_v, sumsq_v, g, b):
    mu = sum_v / _SK
    var = sumsq_v / _SK - mu * mu
    scale = g * lax.rsqrt(var + _EPS)
    shift = b - mu * scale
    return scale, shift


def _pos_stage(pg, sp, w1r, scale1, shift1, a, bb, inv_r, ts):
    tsk = ts * _K
    sp_rep = jnp.broadcast_to(sp.reshape(ts, 1, 16), (ts, _K, 16)).reshape(tsk, 16)
    ptsr = pg - sp_rep
    d2 = jnp.sum(ptsr * ptsr, axis=1, keepdims=True)
    d = jnp.sqrt(d2)
    dw = jax.nn.sigmoid(-a * d + bb)
    dws = jnp.sum(dw.reshape(ts, _K), axis=1, keepdims=True)
    dws = dws + (dws == 0).astype(F32) + 1e-6
    dwn = (dw.reshape(ts, _K) / dws * float(_K)).reshape(tsk, 1)
    ptsn = ptsr * inv_r
    m1p = jnp.dot(ptsn, w1r, preferred_element_type=F32)
    mat1 = jnp.maximum(m1p * scale1 + shift1, 0.0)
    wm1 = mat1 * dwn
    mp1 = jnp.max(wm1.reshape(ts, _K, 16), axis=1, keepdims=True)
    mp1r = jnp.broadcast_to(mp1, (ts, _K, 16)).reshape(tsk, 16)
    return dwn, m1p, mat1, mp1r


def _pass1_body(pg_ref, sp_ref, w1r_ref, scl_ref, s1_ref, q1_ref):
    inv_r = 1.0 / scl_ref[0, 2]
    sp = sp_ref[...]
    pg = pg_ref[...]
    ts = sp.shape[0]
    tsk = ts * _K
    sp_rep = jnp.broadcast_to(sp.reshape(ts, 1, 16), (ts, _K, 16)).reshape(tsk, 16)
    ptsn = (pg - sp_rep) * inv_r
    m1p = jnp.dot(ptsn, w1r_ref[...], preferred_element_type=F32)
    ps = jnp.sum(m1p, axis=0, keepdims=True)
    pq = jnp.sum(m1p * m1p, axis=0, keepdims=True)

    @pl.when(pl.program_id(0) == 0)
    def _():
        s1_ref[...] = jnp.zeros_like(s1_ref)
        q1_ref[...] = jnp.zeros_like(q1_ref)

    s1_ref[...] += ps
    q1_ref[...] += pq


def _pass2_body(pg_ref, sp_ref, w1r_ref, w2a_ref, w2b_ref, scl_ref,
                g1_ref, b1_ref, s1_ref, q1_ref, s2_ref, q2_ref):
    a = scl_ref[0, 0]
    bb = scl_ref[0, 1]
    inv_r = 1.0 / scl_ref[0, 2]
    scale1, shift1 = _affine_from_stats(s1_ref[...], q1_ref[...],
                                        g1_ref[...], b1_ref[...])
    ts = sp_ref.shape[0]
    _, _, mat1, mp1r = _pos_stage(pg_ref[...], sp_ref[...], w1r_ref[...],
                                  scale1, shift1, a, bb, inv_r, ts)
    m2p = (jnp.dot(mat1, w2a_ref[...], preferred_element_type=F32)
           + jnp.dot(mp1r, w2b_ref[...], preferred_element_type=F32))
    ps = jnp.sum(m2p, axis=0, keepdims=True)
    pq = jnp.sum(m2p * m2p, axis=0, keepdims=True)

    @pl.when(pl.program_id(0) == 0)
    def _():
        s2_ref[...] = jnp.zeros_like(s2_ref)
        q2_ref[...] = jnp.zeros_like(q2_ref)

    s2_ref[...] += ps
    q2_ref[...] += pq


def _pass3_body(pg_ref, sp_ref, xg_ref, w1r_ref, w2a_ref, w2b_ref,
                w3a_ref, w3b_ref, wcv_ref, scl_ref,
                g1_ref, b1_ref, g2_ref, b2_ref,
                s1_ref, q1_ref, s2_ref, q2_ref, out_ref):
    a = scl_ref[0, 0]
    bb = scl_ref[0, 1]
    inv_r = 1.0 / scl_ref[0, 2]
    scale1, shift1 = _affine_from_stats(s1_ref[...], q1_ref[...],
                                        g1_ref[...], b1_ref[...])
    scale2, shift2 = _affine_from_stats(s2_ref[...], q2_ref[...],
                                        g2_ref[...], b2_ref[...])
    ts = sp_ref.shape[0]
    tsk = ts * _K
    dwn, _, mat1, mp1r = _pos_stage(pg_ref[...], sp_ref[...], w1r_ref[...],
                                    scale1, shift1, a, bb, inv_r, ts)
    m2p = (jnp.dot(mat1, w2a_ref[...], preferred_element_type=F32)
           + jnp.dot(mp1r, w2b_ref[...], preferred_element_type=F32))
    mat2 = jnp.maximum(m2p * scale2 + shift2, 0.0)
    wm2 = mat2 * dwn
    mp2 = jnp.max(wm2.reshape(ts, _K, 16), axis=1, keepdims=True)
    mp2r = jnp.broadcast_to(mp2, (ts, _K, 16)).reshape(tsk, 16)
    m3p = (jnp.dot(mat2, w3a_ref[...], preferred_element_type=F32)
           + jnp.dot(mp2r, w3b_ref[...], preferred_element_type=F32))
    mat = jnp.maximum(m3p, 0.0) * dwn

    xg = xg_ref[...]
    feats = lax.dot_general(
        mat.reshape(ts, _K, _M), xg.reshape(ts, _K, _C),
        dimension_numbers=(((1,), (1,)), ((0,), (0,))),
        preferred_element_type=F32)
    acc = jnp.zeros((ts, _O), F32)
    for m in range(_M):
        acc = acc + jnp.dot(feats[:, m, :], wcv_ref[m * _C:(m + 1) * _C, :],
                            preferred_element_type=F32)
    out_ref[...] = acc


def _full16(_): return (0, 0)


def _make_pass1(interpret=False):
    grid = (_S // _TS1,)
    return pl.pallas_call(
        _pass1_body,
        grid=grid,
        in_specs=[
            pl.BlockSpec((_TS1 * _K, 16), lambda i: (i, 0)),
            pl.BlockSpec((_TS1, 16), lambda i: (i, 0)),
            pl.BlockSpec((16, 16), _full16),
            pl.BlockSpec((1, 4), _full16),
        ],
        out_specs=[pl.BlockSpec((1, 16), _full16),
                   pl.BlockSpec((1, 16), _full16)],
        out_shape=[jax.ShapeDtypeStruct((1, 16), F32),
                   jax.ShapeDtypeStruct((1, 16), F32)],
        interpret=interpret,
    )


def _make_pass2(interpret=False):
    grid = (_S // _TS1,)
    return pl.pallas_call(
        _pass2_body,
        grid=grid,
        in_specs=[
            pl.BlockSpec((_TS1 * _K, 16), lambda i: (i, 0)),
            pl.BlockSpec((_TS1, 16), lambda i: (i, 0)),
            pl.BlockSpec((16, 16), _full16),
            pl.BlockSpec((16, 16), _full16),
            pl.BlockSpec((16, 16), _full16),
            pl.BlockSpec((1, 4), _full16),
            pl.BlockSpec((1, 16), _full16),
            pl.BlockSpec((1, 16), _full16),
            pl.BlockSpec((1, 16), _full16),
            pl.BlockSpec((1, 16), _full16),
        ],
        out_specs=[pl.BlockSpec((1, 16), _full16),
                   pl.BlockSpec((1, 16), _full16)],
        out_shape=[jax.ShapeDtypeStruct((1, 16), F32),
                   jax.ShapeDtypeStruct((1, 16), F32)],
        interpret=interpret,
    )


def _make_pass3(interpret=False):
    grid = (_S // _TS3,)
    return pl.pallas_call(
        _pass3_body,
        grid=grid,
        in_specs=[
            pl.BlockSpec((_TS3 * _K, 16), lambda i: (i, 0)),
            pl.BlockSpec((_TS3, 16), lambda i: (i, 0)),
            pl.BlockSpec((_TS3 * _K, _C), lambda i: (i, 0)),
            pl.BlockSpec((16, 16), _full16),
            pl.BlockSpec((16, 16), _full16),
            pl.BlockSpec((16, 16), _full16),
            pl.BlockSpec((16, 16), _full16),
            pl.BlockSpec((16, 16), _full16),
            pl.BlockSpec((_M * _C, _O), _full16),
            pl.BlockSpec((1, 4), _full16),
            pl.BlockSpec((1, 16), _full16),
            pl.BlockSpec((1, 16), _full16),
            pl.BlockSpec((1, 16), _full16),
            pl.BlockSpec((1, 16), _full16),
            pl.BlockSpec((1, 16), _full16),
            pl.BlockSpec((1, 16), _full16),
            pl.BlockSpec((1, 16), _full16),
            pl.BlockSpec((1, 16), _full16),
        ],
        out_specs=pl.BlockSpec((_TS3, _O), lambda i: (i, 0)),
        out_shape=jax.ShapeDtypeStruct((_S, _O), F32),
        interpret=interpret,
    )


def _prep_weights(W_fc1, W_fc2, W_fc3, W_cv):
    w1r = jnp.pad(W_fc1, ((0, 0), (0, 13))).T
    w2a = W_fc2[:, :16].T
    w2b = W_fc2[:, 16:].T
    w3a = W_fc3[:, :16].T
    w3b = W_fc3[:, 16:].T
    wcv = jnp.transpose(W_cv, (2, 1, 0)).reshape(_M * _C, _O)
    return w1r, w2a, w2b, w3a, w3b, wcv


def kernel(x, pos, support_points, neighbors_indices, W_fc1, W_fc2, W_fc3,
           g1, b1, g2, b2, alpha, beta, norm_radius, W_cv):
    xt = x[0].T
    pt = jnp.pad(pos[0].T, ((0, 0), (0, 13)))
    spt = jnp.pad(support_points[0].T, ((0, 0), (0, 13)))
    idx = neighbors_indices[0].astype(jnp.int32).reshape(_SK)

    pg = _sc_gather_build(16)(pt, idx)
    xg = _sc_gather_build(_C)(xt, idx)

    w1r, w2a, w2b, w3a, w3b, wcv = _prep_weights(W_fc1, W_fc2, W_fc3, W_cv)
    scl = jnp.concatenate([alpha, beta, norm_radius,
                           jnp.zeros((1,), F32)]).reshape(1, 4)
    g1r, b1r = g1.reshape(1, 16), b1.reshape(1, 16)
    g2r, b2r = g2.reshape(1, 16), b2.reshape(1, 16)

    s1, q1 = _make_pass1()(pg, spt, w1r, scl)
    s2, q2 = _make_pass2()(pg, spt, w1r, w2a, w2b, scl, g1r, b1r, s1, q1)
    out2d = _make_pass3()(pg, spt, xg, w1r, w2a, w2b, w3a, w3b, wcv, scl,
                          g1r, b1r, g2r, b2r, s1, q1, s2, q2)
    return out2d.T[None, :, :]


# speedup vs baseline: 1.7669x; 1.0128x over previous
import functools

import jax
import jax.numpy as jnp
from jax import lax
from jax.experimental import pallas as pl
from jax.experimental.pallas import tpu as pltpu
from jax.experimental.pallas import tpu_sc as plsc

F32 = jnp.float32

_N = 10000
_S = 10000
_K = 16
_C = 128
_O = 128
_M = 16
_SK = _S * _K
_EPS = 1e-5

_NC, _NS = 2, 16
_NW = _NC * _NS
_PER_W = _SK // _NW
_CH = 128
_NFULL = _PER_W // _CH
_TAIL = _PER_W - _NFULL * _CH

_TS1 = 1000
_TS3 = 400


def _sc_gather_build(width):
    mesh = plsc.VectorSubcoreMesh(core_axis_name="c", subcore_axis_name="s")

    nbuf = 4
    lead = 2
    ngrp = (_NFULL + nbuf - 1) // nbuf

    @functools.partial(
        pl.kernel,
        mesh=mesh,
        out_type=jax.ShapeDtypeStruct((_SK, width), F32),
        scratch_types=[
            pltpu.VMEM((_PER_W,), jnp.int32),
            [pltpu.VMEM((_CH, width), F32) for _ in range(nbuf)],
            pltpu.VMEM((_TAIL, width), F32),
            pltpu.SemaphoreType.DMA,
            pltpu.SemaphoreType.DMA,
        ],
        compiler_params=pltpu.CompilerParams(use_tc_tiling_on_sc=False),
    )
    def k(tab_hbm, idx_hbm, out_hbm, idx_v, bufs, rowt_v, semg, semw):
        wid = lax.axis_index("s") * _NC + lax.axis_index("c")
        base = wid * _PER_W

        pltpu.sync_copy(idx_hbm.at[pl.ds(base, _PER_W)], idx_v)

        def start(c, buf):
            pltpu.async_copy(tab_hbm.at[idx_v.at[pl.ds(c * _CH, _CH)]],
                             buf, semg)

        def wait_gather(c, buf):
            pltpu.make_async_copy(tab_hbm.at[idx_v.at[pl.ds(c * _CH, _CH)]],
                                  buf, semg).wait()

        def wait_one_writeout(buf):
            pltpu.make_async_copy(buf, out_hbm.at[pl.ds(base, _CH)],
                                  semw).wait()

        for c0 in range(lead):
            start(c0, bufs[c0 % nbuf])

        def body(j, carry):
            for b in range(nbuf):
                cc = nbuf * j + b
                bufc = bufs[b]
                bufn = bufs[(b + lead) % nbuf]

                @pl.when(cc < _NFULL)
                def _():
                    wait_gather(cc, bufc)
                    pltpu.async_copy(bufc,
                                     out_hbm.at[pl.ds(base + cc * _CH, _CH)],
                                     semw)

                @pl.when(jnp.logical_and(cc >= nbuf - lead,
                                         cc + lead < _NFULL))
                def _():
                    wait_one_writeout(bufn)

                @pl.when(cc + lead < _NFULL)
                def _():
                    start(cc + lead, bufn)
            return carry

        lax.fori_loop(0, ngrp, body, 0)

        pltpu.async_copy(tab_hbm.at[idx_v.at[pl.ds(_NFULL * _CH, _TAIL)]],
                         rowt_v, semg).wait()
        pltpu.sync_copy(rowt_v, out_hbm.at[pl.ds(base + _NFULL * _CH, _TAIL)])
        for _ in range(nbuf):
            wait_one_writeout(bufs[0])

    return k


def _affine_from_stats(sum_v, sumsq_v, g, b):
    mu = sum_v / _SK
    var = sumsq_v / _SK - mu * mu
    scale = g * lax.rsqrt(var + _EPS)
    shift = b - mu * scale
    return scale, shift


def _pos_stage(pg, sp, w1r, scale1, shift1, a, bb, inv_r, ts):
    tsk = ts * _K
    sp_rep = jnp.broadcast_to(sp.reshape(ts, 1, 16), (ts, _K, 16)).reshape(tsk, 16)
    ptsr = pg - sp_rep
    d2 = jnp.sum(ptsr * ptsr, axis=1, keepdims=True)
    d = jnp.sqrt(d2)
    dw = jax.nn.sigmoid(-a * d + bb)
    dws = jnp.sum(dw.reshape(ts, _K), axis=1, keepdims=True)
    dws = dws + (dws == 0).astype(F32) + 1e-6
    dwn = (dw.reshape(ts, _K) / dws * float(_K)).reshape(tsk, 1)
    ptsn = ptsr * inv_r
    m1p = jnp.dot(ptsn, w1r, preferred_element_type=F32)
    mat1 = jnp.maximum(m1p * scale1 + shift1, 0.0)
    wm1 = mat1 * dwn
    mp1 = jnp.max(wm1.reshape(ts, _K, 16), axis=1, keepdims=True)
    mp1r = jnp.broadcast_to(mp1, (ts, _K, 16)).reshape(tsk, 16)
    return dwn, m1p, mat1, mp1r


def _pass1_body(pg_ref, sp_ref, w1r_ref, scl_ref, s1_ref, q1_ref):
    inv_r = 1.0 / scl_ref[0, 2]
    sp = sp_ref[...]
    pg = pg_ref[...]
    ts = sp.shape[0]
    tsk = ts * _K
    sp_rep = jnp.broadcast_to(sp.reshape(ts, 1, 16), (ts, _K, 16)).reshape(tsk, 16)
    ptsn = (pg - sp_rep) * inv_r
    m1p = jnp.dot(ptsn, w1r_ref[...], preferred_element_type=F32)
    ps = jnp.sum(m1p, axis=0, keepdims=True)
    pq = jnp.sum(m1p * m1p, axis=0, keepdims=True)

    @pl.when(pl.program_id(0) == 0)
    def _():
        s1_ref[...] = jnp.zeros_like(s1_ref)
        q1_ref[...] = jnp.zeros_like(q1_ref)

    s1_ref[...] += ps
    q1_ref[...] += pq


def _pass2_body(pg_ref, sp_ref, w1r_ref, w2a_ref, w2b_ref, scl_ref,
                g1_ref, b1_ref, s1_ref, q1_ref, s2_ref, q2_ref):
    a = scl_ref[0, 0]
    bb = scl_ref[0, 1]
    inv_r = 1.0 / scl_ref[0, 2]
    scale1, shift1 = _affine_from_stats(s1_ref[...], q1_ref[...],
                                        g1_ref[...], b1_ref[...])
    ts = sp_ref.shape[0]
    _, _, mat1, mp1r = _pos_stage(pg_ref[...], sp_ref[...], w1r_ref[...],
                                  scale1, shift1, a, bb, inv_r, ts)
    m2p = (jnp.dot(mat1, w2a_ref[...], preferred_element_type=F32)
           + jnp.dot(mp1r, w2b_ref[...], preferred_element_type=F32))
    ps = jnp.sum(m2p, axis=0, keepdims=True)
    pq = jnp.sum(m2p * m2p, axis=0, keepdims=True)

    @pl.when(pl.program_id(0) == 0)
    def _():
        s2_ref[...] = jnp.zeros_like(s2_ref)
        q2_ref[...] = jnp.zeros_like(q2_ref)

    s2_ref[...] += ps
    q2_ref[...] += pq


def _pass3_body(pg_ref, sp_ref, xg_ref, w1r_ref, w2a_ref, w2b_ref,
                w3a_ref, w3b_ref, wcv_ref, scl_ref,
                g1_ref, b1_ref, g2_ref, b2_ref,
                s1_ref, q1_ref, s2_ref, q2_ref, out_ref):
    a = scl_ref[0, 0]
    bb = scl_ref[0, 1]
    inv_r = 1.0 / scl_ref[0, 2]
    scale1, shift1 = _affine_from_stats(s1_ref[...], q1_ref[...],
                                        g1_ref[...], b1_ref[...])
    scale2, shift2 = _affine_from_stats(s2_ref[...], q2_ref[...],
                                        g2_ref[...], b2_ref[...])
    ts = sp_ref.shape[0]
    tsk = ts * _K
    dwn, _, mat1, mp1r = _pos_stage(pg_ref[...], sp_ref[...], w1r_ref[...],
                                    scale1, shift1, a, bb, inv_r, ts)
    m2p = (jnp.dot(mat1, w2a_ref[...], preferred_element_type=F32)
           + jnp.dot(mp1r, w2b_ref[...], preferred_element_type=F32))
    mat2 = jnp.maximum(m2p * scale2 + shift2, 0.0)
    wm2 = mat2 * dwn
    mp2 = jnp.max(wm2.reshape(ts, _K, 16), axis=1, keepdims=True)
    mp2r = jnp.broadcast_to(mp2, (ts, _K, 16)).reshape(tsk, 16)
    m3p = (jnp.dot(mat2, w3a_ref[...], preferred_element_type=F32)
           + jnp.dot(mp2r, w3b_ref[...], preferred_element_type=F32))
    mat = jnp.maximum(m3p, 0.0) * dwn

    xg = xg_ref[...]
    feats = lax.dot_general(
        mat.reshape(ts, _K, _M), xg.reshape(ts, _K, _C),
        dimension_numbers=(((1,), (1,)), ((0,), (0,))),
        preferred_element_type=F32)
    acc = jnp.zeros((ts, _O), F32)
    for m in range(_M):
        acc = acc + jnp.dot(feats[:, m, :], wcv_ref[m * _C:(m + 1) * _C, :],
                            preferred_element_type=F32)
    out_ref[...] = acc


def _full16(_): return (0, 0)


def _make_pass1(interpret=False):
    grid = (_S // _TS1,)
    return pl.pallas_call(
        _pass1_body,
        grid=grid,
        in_specs=[
            pl.BlockSpec((_TS1 * _K, 16), lambda i: (i, 0)),
            pl.BlockSpec((_TS1, 16), lambda i: (i, 0)),
            pl.BlockSpec((16, 16), _full16),
            pl.BlockSpec((1, 4), _full16),
        ],
        out_specs=[pl.BlockSpec((1, 16), _full16),
                   pl.BlockSpec((1, 16), _full16)],
        out_shape=[jax.ShapeDtypeStruct((1, 16), F32),
                   jax.ShapeDtypeStruct((1, 16), F32)],
        interpret=interpret,
    )


def _make_pass2(interpret=False):
    grid = (_S // _TS1,)
    return pl.pallas_call(
        _pass2_body,
        grid=grid,
        in_specs=[
            pl.BlockSpec((_TS1 * _K, 16), lambda i: (i, 0)),
            pl.BlockSpec((_TS1, 16), lambda i: (i, 0)),
            pl.BlockSpec((16, 16), _full16),
            pl.BlockSpec((16, 16), _full16),
            pl.BlockSpec((16, 16), _full16),
            pl.BlockSpec((1, 4), _full16),
            pl.BlockSpec((1, 16), _full16),
            pl.BlockSpec((1, 16), _full16),
            pl.BlockSpec((1, 16), _full16),
            pl.BlockSpec((1, 16), _full16),
        ],
        out_specs=[pl.BlockSpec((1, 16), _full16),
                   pl.BlockSpec((1, 16), _full16)],
        out_shape=[jax.ShapeDtypeStruct((1, 16), F32),
                   jax.ShapeDtypeStruct((1, 16), F32)],
        interpret=interpret,
    )


def _make_pass3(interpret=False):
    grid = (_S // _TS3,)
    return pl.pallas_call(
        _pass3_body,
        grid=grid,
        in_specs=[
            pl.BlockSpec((_TS3 * _K, 16), lambda i: (i, 0)),
            pl.BlockSpec((_TS3, 16), lambda i: (i, 0)),
            pl.BlockSpec((_TS3 * _K, _C), lambda i: (i, 0)),
            pl.BlockSpec((16, 16), _full16),
            pl.BlockSpec((16, 16), _full16),
            pl.BlockSpec((16, 16), _full16),
            pl.BlockSpec((16, 16), _full16),
            pl.BlockSpec((16, 16), _full16),
            pl.BlockSpec((_M * _C, _O), _full16),
            pl.BlockSpec((1, 4), _full16),
            pl.BlockSpec((1, 16), _full16),
            pl.BlockSpec((1, 16), _full16),
            pl.BlockSpec((1, 16), _full16),
            pl.BlockSpec((1, 16), _full16),
            pl.BlockSpec((1, 16), _full16),
            pl.BlockSpec((1, 16), _full16),
            pl.BlockSpec((1, 16), _full16),
            pl.BlockSpec((1, 16), _full16),
        ],
        out_specs=pl.BlockSpec((_TS3, _O), lambda i: (i, 0)),
        out_shape=jax.ShapeDtypeStruct((_S, _O), F32),
        interpret=interpret,
    )


def _prep_weights(W_fc1, W_fc2, W_fc3, W_cv):
    w1r = jnp.pad(W_fc1, ((0, 0), (0, 13))).T
    w2a = W_fc2[:, :16].T
    w2b = W_fc2[:, 16:].T
    w3a = W_fc3[:, :16].T
    w3b = W_fc3[:, 16:].T
    wcv = jnp.transpose(W_cv, (2, 1, 0)).reshape(_M * _C, _O)
    return w1r, w2a, w2b, w3a, w3b, wcv


def kernel(x, pos, support_points, neighbors_indices, W_fc1, W_fc2, W_fc3,
           g1, b1, g2, b2, alpha, beta, norm_radius, W_cv):
    xt = x[0].T
    pt = jnp.pad(pos[0].T, ((0, 0), (0, 13)))
    spt = jnp.pad(support_points[0].T, ((0, 0), (0, 13)))
    idx = neighbors_indices[0].astype(jnp.int32).reshape(_SK)

    pg = _sc_gather_build(16)(pt, idx)
    xg = _sc_gather_build(_C)(xt, idx)

    w1r, w2a, w2b, w3a, w3b, wcv = _prep_weights(W_fc1, W_fc2, W_fc3, W_cv)
    scl = jnp.concatenate([alpha, beta, norm_radius,
                           jnp.zeros((1,), F32)]).reshape(1, 4)
    g1r, b1r = g1.reshape(1, 16), b1.reshape(1, 16)
    g2r, b2r = g2.reshape(1, 16), b2.reshape(1, 16)

    s1, q1 = _make_pass1()(pg, spt, w1r, scl)
    s2, q2 = _make_pass2()(pg, spt, w1r, w2a, w2b, scl, g1r, b1r, s1, q1)
    out2d = _make_pass3()(pg, spt, xg, w1r, w2a, w2b, w3a, w3b, wcv, scl,
                          g1r, b1r, g2r, b2r, s1, q1, s2, q2)
    return out2d.T[None, :, :]


# speedup vs baseline: 2.3285x; 1.3179x over previous
import functools

import jax
import jax.numpy as jnp
from jax import lax
from jax.experimental import pallas as pl
from jax.experimental.pallas import tpu as pltpu
from jax.experimental.pallas import tpu_sc as plsc

F32 = jnp.float32

_N = 10000
_S = 10000
_K = 16
_C = 128
_O = 128
_M = 16
_SK = _S * _K
_EPS = 1e-5

_NC, _NS = 2, 16
_NW = _NC * _NS
_PER_W = _SK // _NW
_CH = 128
_NFULL = _PER_W // _CH
_TAIL = _PER_W - _NFULL * _CH

_TS1 = 1000
_TS3 = 400


def _sc_gather_build(width):
    mesh = plsc.VectorSubcoreMesh(core_axis_name="c", subcore_axis_name="s")

    nbuf = 4
    lead = 2
    ngrp = (_NFULL + nbuf - 1) // nbuf

    @functools.partial(
        pl.kernel,
        mesh=mesh,
        out_type=jax.ShapeDtypeStruct((_SK, width), F32),
        scratch_types=[
            pltpu.VMEM((_PER_W,), jnp.int32),
            [pltpu.VMEM((_CH, width), F32) for _ in range(nbuf)],
            pltpu.VMEM((_TAIL, width), F32),
            pltpu.SemaphoreType.DMA,
            pltpu.SemaphoreType.DMA,
        ],
        compiler_params=pltpu.CompilerParams(use_tc_tiling_on_sc=False),
    )
    def k(tab_hbm, idx_hbm, out_hbm, idx_v, bufs, rowt_v, semg, semw):
        wid = lax.axis_index("s") * _NC + lax.axis_index("c")
        base = wid * _PER_W

        pltpu.sync_copy(idx_hbm.at[pl.ds(base, _PER_W)], idx_v)

        def start(c, buf):
            pltpu.async_copy(tab_hbm.at[idx_v.at[pl.ds(c * _CH, _CH)]],
                             buf, semg)

        def wait_gather(c, buf):
            pltpu.make_async_copy(tab_hbm.at[idx_v.at[pl.ds(c * _CH, _CH)]],
                                  buf, semg).wait()

        def wait_one_writeout(buf):
            pltpu.make_async_copy(buf, out_hbm.at[pl.ds(base, _CH)],
                                  semw).wait()

        for c0 in range(lead):
            start(c0, bufs[c0 % nbuf])

        def body(j, carry):
            for b in range(nbuf):
                cc = nbuf * j + b
                bufc = bufs[b]
                bufn = bufs[(b + lead) % nbuf]

                @pl.when(cc < _NFULL)
                def _():
                    wait_gather(cc, bufc)
                    pltpu.async_copy(bufc,
                                     out_hbm.at[pl.ds(base + cc * _CH, _CH)],
                                     semw)

                @pl.when(jnp.logical_and(cc >= nbuf - lead,
                                         cc + lead < _NFULL))
                def _():
                    wait_one_writeout(bufn)

                @pl.when(cc + lead < _NFULL)
                def _():
                    start(cc + lead, bufn)
            return carry

        lax.fori_loop(0, ngrp, body, 0)

        pltpu.async_copy(tab_hbm.at[idx_v.at[pl.ds(_NFULL * _CH, _TAIL)]],
                         rowt_v, semg).wait()
        pltpu.sync_copy(rowt_v, out_hbm.at[pl.ds(base + _NFULL * _CH, _TAIL)])
        for _ in range(nbuf):
            wait_one_writeout(bufs[0])

    return k


def _affine_from_stats(sum_v, sumsq_v, g, b):
    mu = sum_v / _SK
    var = sumsq_v / _SK - mu * mu
    scale = g * lax.rsqrt(var + _EPS)
    shift = b - mu * scale
    return scale, shift


def _pos_stage(pg, sp, w1r, scale1, shift1, a, bb, inv_r, ts):
    tsk = ts * _K
    sp_rep = jnp.broadcast_to(sp.reshape(ts, 1, 16), (ts, _K, 16)).reshape(tsk, 16)
    ptsr = pg - sp_rep
    d2 = jnp.sum(ptsr * ptsr, axis=1, keepdims=True)
    d = jnp.sqrt(d2)
    dw = jax.nn.sigmoid(-a * d + bb)
    dws = jnp.sum(dw.reshape(ts, _K), axis=1, keepdims=True)
    dws = dws + (dws == 0).astype(F32) + 1e-6
    dwn = (dw.reshape(ts, _K) / dws * float(_K)).reshape(tsk, 1)
    ptsn = ptsr * inv_r
    m1p = jnp.dot(ptsn, w1r, preferred_element_type=F32)
    mat1 = jnp.maximum(m1p * scale1 + shift1, 0.0)
    wm1 = mat1 * dwn
    mp1 = jnp.max(wm1.reshape(ts, _K, 16), axis=1, keepdims=True)
    mp1r = jnp.broadcast_to(mp1, (ts, _K, 16)).reshape(tsk, 16)
    return dwn, m1p, mat1, mp1r


def _pass1_body(pg_ref, sp_ref, w1r_ref, scl_ref, s1_ref, q1_ref):
    inv_r = 1.0 / scl_ref[0, 2]
    sp = sp_ref[...]
    pg = pg_ref[...]
    ts = sp.shape[0]
    tsk = ts * _K
    sp_rep = jnp.broadcast_to(sp.reshape(ts, 1, 16), (ts, _K, 16)).reshape(tsk, 16)
    ptsn = (pg - sp_rep) * inv_r
    m1p = jnp.dot(ptsn, w1r_ref[...], preferred_element_type=F32)
    ps = jnp.sum(m1p, axis=0, keepdims=True)
    pq = jnp.sum(m1p * m1p, axis=0, keepdims=True)

    @pl.when(pl.program_id(0) == 0)
    def _():
        s1_ref[...] = jnp.zeros_like(s1_ref)
        q1_ref[...] = jnp.zeros_like(q1_ref)

    s1_ref[...] += ps
    q1_ref[...] += pq


def _pass2_body(pg_ref, sp_ref, w1r_ref, w2a_ref, w2b_ref, scl_ref,
                g1_ref, b1_ref, s1_ref, q1_ref, s2_ref, q2_ref, mz_ref):
    a = scl_ref[0, 0]
    bb = scl_ref[0, 1]
    inv_r = 1.0 / scl_ref[0, 2]
    scale1, shift1 = _affine_from_stats(s1_ref[...], q1_ref[...],
                                        g1_ref[...], b1_ref[...])
    ts = sp_ref.shape[0]
    tsk = ts * _K
    dwn, _, mat1, mp1r = _pos_stage(pg_ref[...], sp_ref[...], w1r_ref[...],
                                    scale1, shift1, a, bb, inv_r, ts)
    m2p = (jnp.dot(mat1, w2a_ref[...], preferred_element_type=F32)
           + jnp.dot(mp1r, w2b_ref[...], preferred_element_type=F32))
    ps = jnp.sum(m2p, axis=0, keepdims=True)
    pq = jnp.sum(m2p * m2p, axis=0, keepdims=True)
    mz_ref[...] = jnp.concatenate(
        [m2p, jnp.broadcast_to(dwn, (tsk, 16))], axis=1)

    @pl.when(pl.program_id(0) == 0)
    def _():
        s2_ref[...] = jnp.zeros_like(s2_ref)
        q2_ref[...] = jnp.zeros_like(q2_ref)

    s2_ref[...] += ps
    q2_ref[...] += pq


def _pass3_body(mz_ref, xg_ref, w3a_ref, w3b_ref, wcv_ref,
                g2_ref, b2_ref, s2_ref, q2_ref, out_ref):
    scale2, shift2 = _affine_from_stats(s2_ref[...], q2_ref[...],
                                        g2_ref[...], b2_ref[...])
    tsk = mz_ref.shape[0]
    ts = tsk // _K
    mz = mz_ref[...]
    m2p = mz[:, :16]
    dwn = mz[:, 16:17]
    mat2 = jnp.maximum(m2p * scale2 + shift2, 0.0)
    wm2 = mat2 * dwn
    mp2 = jnp.max(wm2.reshape(ts, _K, 16), axis=1, keepdims=True)
    mp2r = jnp.broadcast_to(mp2, (ts, _K, 16)).reshape(tsk, 16)
    m3p = (jnp.dot(mat2, w3a_ref[...], preferred_element_type=F32)
           + jnp.dot(mp2r, w3b_ref[...], preferred_element_type=F32))
    mat = jnp.maximum(m3p, 0.0) * dwn

    xg = xg_ref[...]
    feats = lax.dot_general(
        mat.reshape(ts, _K, _M), xg.reshape(ts, _K, _C),
        dimension_numbers=(((1,), (1,)), ((0,), (0,))),
        preferred_element_type=F32)
    acc = jnp.zeros((ts, _O), F32)
    for m in range(_M):
        acc = acc + jnp.dot(feats[:, m, :], wcv_ref[m * _C:(m + 1) * _C, :],
                            preferred_element_type=F32)
    out_ref[...] = acc


def _full16(_): return (0, 0)


def _make_pass1(interpret=False):
    grid = (_S // _TS1,)
    return pl.pallas_call(
        _pass1_body,
        grid=grid,
        in_specs=[
            pl.BlockSpec((_TS1 * _K, 16), lambda i: (i, 0)),
            pl.BlockSpec((_TS1, 16), lambda i: (i, 0)),
            pl.BlockSpec((16, 16), _full16),
            pl.BlockSpec((1, 4), _full16),
        ],
        out_specs=[pl.BlockSpec((1, 16), _full16),
                   pl.BlockSpec((1, 16), _full16)],
        out_shape=[jax.ShapeDtypeStruct((1, 16), F32),
                   jax.ShapeDtypeStruct((1, 16), F32)],
        interpret=interpret,
    )


def _make_pass2(interpret=False):
    grid = (_S // _TS1,)
    return pl.pallas_call(
        _pass2_body,
        grid=grid,
        in_specs=[
            pl.BlockSpec((_TS1 * _K, 16), lambda i: (i, 0)),
            pl.BlockSpec((_TS1, 16), lambda i: (i, 0)),
            pl.BlockSpec((16, 16), _full16),
            pl.BlockSpec((16, 16), _full16),
            pl.BlockSpec((16, 16), _full16),
            pl.BlockSpec((1, 4), _full16),
            pl.BlockSpec((1, 16), _full16),
            pl.BlockSpec((1, 16), _full16),
            pl.BlockSpec((1, 16), _full16),
            pl.BlockSpec((1, 16), _full16),
        ],
        out_specs=[pl.BlockSpec((1, 16), _full16),
                   pl.BlockSpec((1, 16), _full16),
                   pl.BlockSpec((_TS1 * _K, 32), lambda i: (i, 0))],
        out_shape=[jax.ShapeDtypeStruct((1, 16), F32),
                   jax.ShapeDtypeStruct((1, 16), F32),
                   jax.ShapeDtypeStruct((_SK, 32), F32)],
        interpret=interpret,
    )


def _make_pass3(interpret=False):
    grid = (_S // _TS3,)
    return pl.pallas_call(
        _pass3_body,
        grid=grid,
        in_specs=[
            pl.BlockSpec((_TS3 * _K, 32), lambda i: (i, 0)),
            pl.BlockSpec((_TS3 * _K, _C), lambda i: (i, 0)),
            pl.BlockSpec((16, 16), _full16),
            pl.BlockSpec((16, 16), _full16),
            pl.BlockSpec((_M * _C, _O), _full16),
            pl.BlockSpec((1, 16), _full16),
            pl.BlockSpec((1, 16), _full16),
            pl.BlockSpec((1, 16), _full16),
            pl.BlockSpec((1, 16), _full16),
        ],
        out_specs=pl.BlockSpec((_TS3, _O), lambda i: (i, 0)),
        out_shape=jax.ShapeDtypeStruct((_S, _O), F32),
        interpret=interpret,
    )


def _prep_weights(W_fc1, W_fc2, W_fc3, W_cv):
    w1r = jnp.pad(W_fc1, ((0, 0), (0, 13))).T
    w2a = W_fc2[:, :16].T
    w2b = W_fc2[:, 16:].T
    w3a = W_fc3[:, :16].T
    w3b = W_fc3[:, 16:].T
    wcv = jnp.transpose(W_cv, (2, 1, 0)).reshape(_M * _C, _O)
    return w1r, w2a, w2b, w3a, w3b, wcv


def kernel(x, pos, support_points, neighbors_indices, W_fc1, W_fc2, W_fc3,
           g1, b1, g2, b2, alpha, beta, norm_radius, W_cv):
    xt = x[0].T
    pt = jnp.pad(pos[0].T, ((0, 0), (0, 13)))
    spt = jnp.pad(support_points[0].T, ((0, 0), (0, 13)))
    idx = neighbors_indices[0].astype(jnp.int32).reshape(_SK)

    pg = _sc_gather_build(16)(pt, idx)
    xg = _sc_gather_build(_C)(xt, idx)

    w1r, w2a, w2b, w3a, w3b, wcv = _prep_weights(W_fc1, W_fc2, W_fc3, W_cv)
    scl = jnp.concatenate([alpha, beta, norm_radius,
                           jnp.zeros((1,), F32)]).reshape(1, 4)
    g1r, b1r = g1.reshape(1, 16), b1.reshape(1, 16)
    g2r, b2r = g2.reshape(1, 16), b2.reshape(1, 16)

    s1, q1 = _make_pass1()(pg, spt, w1r, scl)
    s2, q2, mz = _make_pass2()(pg, spt, w1r, w2a, w2b, scl, g1r, b1r, s1, q1)
    out2d = _make_pass3()(mz, xg, w3a, w3b, wcv, g2r, b2r, s2, q2)
    return out2d.T[None, :, :]


# speedup vs baseline: 2.3291x; 1.0002x over previous
import functools

import jax
import jax.numpy as jnp
from jax import lax
from jax.experimental import pallas as pl
from jax.experimental.pallas import tpu as pltpu
from jax.experimental.pallas import tpu_sc as plsc

F32 = jnp.float32

_N = 10000
_S = 10000
_K = 16
_C = 128
_O = 128
_M = 16
_SK = _S * _K
_EPS = 1e-5

_NC, _NS = 2, 16
_NW = _NC * _NS
_PER_W = _SK // _NW
_CH = 128
_NFULL = _PER_W // _CH
_TAIL = _PER_W - _NFULL * _CH

_TS1 = 1000
_TS3 = 400


def _sc_gather_build(width):
    mesh = plsc.VectorSubcoreMesh(core_axis_name="c", subcore_axis_name="s")

    nbuf = 4
    lead = 2
    ngrp = (_NFULL + nbuf - 1) // nbuf

    @functools.partial(
        pl.kernel,
        mesh=mesh,
        out_type=jax.ShapeDtypeStruct((_SK, width), F32),
        scratch_types=[
            pltpu.VMEM((_PER_W,), jnp.int32),
            [pltpu.VMEM((_CH, width), F32) for _ in range(nbuf)],
            pltpu.VMEM((_TAIL, width), F32),
            pltpu.SemaphoreType.DMA,
            pltpu.SemaphoreType.DMA,
        ],
        compiler_params=pltpu.CompilerParams(use_tc_tiling_on_sc=False),
    )
    def k(tab_hbm, idx_hbm, out_hbm, idx_v, bufs, rowt_v, semg, semw):
        wid = lax.axis_index("s") * _NC + lax.axis_index("c")
        base = wid * _PER_W

        pltpu.sync_copy(idx_hbm.at[pl.ds(base, _PER_W)], idx_v)

        def start(c, buf):
            pltpu.async_copy(tab_hbm.at[idx_v.at[pl.ds(c * _CH, _CH)]],
                             buf, semg)

        def wait_gather(c, buf):
            pltpu.make_async_copy(tab_hbm.at[idx_v.at[pl.ds(c * _CH, _CH)]],
                                  buf, semg).wait()

        def wait_one_writeout(buf):
            pltpu.make_async_copy(buf, out_hbm.at[pl.ds(base, _CH)],
                                  semw).wait()

        for c0 in range(lead):
            start(c0, bufs[c0 % nbuf])

        def body(j, carry):
            for b in range(nbuf):
                cc = nbuf * j + b
                bufc = bufs[b]
                bufn = bufs[(b + lead) % nbuf]

                @pl.when(cc < _NFULL)
                def _():
                    wait_gather(cc, bufc)
                    pltpu.async_copy(bufc,
                                     out_hbm.at[pl.ds(base + cc * _CH, _CH)],
                                     semw)

                @pl.when(jnp.logical_and(cc >= nbuf - lead,
                                         cc + lead < _NFULL))
                def _():
                    wait_one_writeout(bufn)

                @pl.when(cc + lead < _NFULL)
                def _():
                    start(cc + lead, bufn)
            return carry

        lax.fori_loop(0, ngrp, body, 0)

        pltpu.async_copy(tab_hbm.at[idx_v.at[pl.ds(_NFULL * _CH, _TAIL)]],
                         rowt_v, semg).wait()
        pltpu.sync_copy(rowt_v, out_hbm.at[pl.ds(base + _NFULL * _CH, _TAIL)])
        for _ in range(nbuf):
            wait_one_writeout(bufs[0])

    return k


def _affine_from_stats(sum_v, sumsq_v, g, b):
    mu = sum_v / _SK
    var = sumsq_v / _SK - mu * mu
    scale = g * lax.rsqrt(var + _EPS)
    shift = b - mu * scale
    return scale, shift


def _pos_stage(pg, sp, w1r, scale1, shift1, a, bb, inv_r, ts):
    tsk = ts * _K
    sp_rep = jnp.broadcast_to(sp.reshape(ts, 1, 16), (ts, _K, 16)).reshape(tsk, 16)
    ptsr = pg - sp_rep
    d2 = jnp.sum(ptsr * ptsr, axis=1, keepdims=True)
    d = jnp.sqrt(d2)
    dw = jax.nn.sigmoid(-a * d + bb)
    dws = jnp.sum(dw.reshape(ts, _K), axis=1, keepdims=True)
    dws = dws + (dws == 0).astype(F32) + 1e-6
    dwn = (dw.reshape(ts, _K) / dws * float(_K)).reshape(tsk, 1)
    ptsn = ptsr * inv_r
    m1p = jnp.dot(ptsn, w1r, preferred_element_type=F32)
    mat1 = jnp.maximum(m1p * scale1 + shift1, 0.0)
    wm1 = mat1 * dwn
    mp1 = jnp.max(wm1.reshape(ts, _K, 16), axis=1, keepdims=True)
    mp1r = jnp.broadcast_to(mp1, (ts, _K, 16)).reshape(tsk, 16)
    return dwn, m1p, mat1, mp1r


def _pass1_body(pg_ref, sp_ref, w1r_ref, scl_ref, s1_ref, q1_ref):
    inv_r = 1.0 / scl_ref[0, 2]
    sp = sp_ref[...]
    pg = pg_ref[...]
    ts = sp.shape[0]
    tsk = ts * _K
    sp_rep = jnp.broadcast_to(sp.reshape(ts, 1, 16), (ts, _K, 16)).reshape(tsk, 16)
    ptsn = (pg - sp_rep) * inv_r
    m1p = jnp.dot(ptsn, w1r_ref[...], preferred_element_type=F32)
    ps = jnp.sum(m1p, axis=0, keepdims=True)
    pq = jnp.sum(m1p * m1p, axis=0, keepdims=True)

    @pl.when(pl.program_id(0) == 0)
    def _():
        s1_ref[...] = jnp.zeros_like(s1_ref)
        q1_ref[...] = jnp.zeros_like(q1_ref)

    s1_ref[...] += ps
    q1_ref[...] += pq


def _pass2_body(pg_ref, sp_ref, w1r_ref, w2a_ref, w2b_ref, scl_ref,
                g1_ref, b1_ref, s1_ref, q1_ref, s2_ref, q2_ref, mz_ref):
    a = scl_ref[0, 0]
    bb = scl_ref[0, 1]
    inv_r = 1.0 / scl_ref[0, 2]
    scale1, shift1 = _affine_from_stats(s1_ref[...], q1_ref[...],
                                        g1_ref[...], b1_ref[...])
    ts = sp_ref.shape[0]
    tsk = ts * _K
    dwn, _, mat1, mp1r = _pos_stage(pg_ref[...], sp_ref[...], w1r_ref[...],
                                    scale1, shift1, a, bb, inv_r, ts)
    m2p = (jnp.dot(mat1, w2a_ref[...], preferred_element_type=F32)
           + jnp.dot(mp1r, w2b_ref[...], preferred_element_type=F32))
    ps = jnp.sum(m2p, axis=0, keepdims=True)
    pq = jnp.sum(m2p * m2p, axis=0, keepdims=True)
    mz_ref[...] = jnp.concatenate(
        [m2p, jnp.broadcast_to(dwn, (tsk, 16))], axis=1)

    @pl.when(pl.program_id(0) == 0)
    def _():
        s2_ref[...] = jnp.zeros_like(s2_ref)
        q2_ref[...] = jnp.zeros_like(q2_ref)

    s2_ref[...] += ps
    q2_ref[...] += pq


def _pass3_body(mz_ref, xg_ref, w3a_ref, w3b_ref, wcv_ref,
                g2_ref, b2_ref, s2_ref, q2_ref, out_ref):
    scale2, shift2 = _affine_from_stats(s2_ref[...], q2_ref[...],
                                        g2_ref[...], b2_ref[...])
    tsk = mz_ref.shape[0]
    ts = tsk // _K
    mz = mz_ref[...]
    m2p = mz[:, :16]
    dwn = mz[:, 16:17]
    mat2 = jnp.maximum(m2p * scale2 + shift2, 0.0)
    wm2 = mat2 * dwn
    mp2 = jnp.max(wm2.reshape(ts, _K, 16), axis=1, keepdims=True)
    mp2r = jnp.broadcast_to(mp2, (ts, _K, 16)).reshape(tsk, 16)
    m3p = (jnp.dot(mat2, w3a_ref[...], preferred_element_type=F32)
           + jnp.dot(mp2r, w3b_ref[...], preferred_element_type=F32))
    mat = jnp.maximum(m3p, 0.0) * dwn

    xg = xg_ref[...]
    feats = lax.dot_general(
        mat.reshape(ts, _K, _M), xg.reshape(ts, _K, _C),
        dimension_numbers=(((1,), (1,)), ((0,), (0,))),
        preferred_element_type=F32)
    acc = jnp.zeros((ts, _O), F32)
    for m in range(_M):
        acc = acc + jnp.dot(feats[:, m, :], wcv_ref[m * _C:(m + 1) * _C, :],
                            preferred_element_type=F32)
    out_ref[...] = acc


def _full16(_): return (0, 0)


def _make_pass1(interpret=False):
    grid = (_S // _TS1,)
    return pl.pallas_call(
        _pass1_body,
        grid=grid,
        in_specs=[
            pl.BlockSpec((_TS1 * _K, 16), lambda i: (i, 0)),
            pl.BlockSpec((_TS1, 16), lambda i: (i, 0)),
            pl.BlockSpec((16, 16), _full16),
            pl.BlockSpec((1, 4), _full16),
        ],
        out_specs=[pl.BlockSpec((1, 16), _full16),
                   pl.BlockSpec((1, 16), _full16)],
        out_shape=[jax.ShapeDtypeStruct((1, 16), F32),
                   jax.ShapeDtypeStruct((1, 16), F32)],
        interpret=interpret,
    )


def _make_pass2(interpret=False):
    grid = (_S // _TS1,)
    return pl.pallas_call(
        _pass2_body,
        grid=grid,
        in_specs=[
            pl.BlockSpec((_TS1 * _K, 16), lambda i: (i, 0)),
            pl.BlockSpec((_TS1, 16), lambda i: (i, 0)),
            pl.BlockSpec((16, 16), _full16),
            pl.BlockSpec((16, 16), _full16),
            pl.BlockSpec((16, 16), _full16),
            pl.BlockSpec((1, 4), _full16),
            pl.BlockSpec((1, 16), _full16),
            pl.BlockSpec((1, 16), _full16),
            pl.BlockSpec((1, 16), _full16),
            pl.BlockSpec((1, 16), _full16),
        ],
        out_specs=[pl.BlockSpec((1, 16), _full16),
                   pl.BlockSpec((1, 16), _full16),
                   pl.BlockSpec((_TS1 * _K, 32), lambda i: (i, 0))],
        out_shape=[jax.ShapeDtypeStruct((1, 16), F32),
                   jax.ShapeDtypeStruct((1, 16), F32),
                   jax.ShapeDtypeStruct((_SK, 32), F32)],
        interpret=interpret,
    )


def _make_pass3(interpret=False):
    grid = (_S // _TS3,)
    return pl.pallas_call(
        _pass3_body,
        grid=grid,
        in_specs=[
            pl.BlockSpec((_TS3 * _K, 32), lambda i: (i, 0)),
            pl.BlockSpec((_TS3 * _K, _C), lambda i: (i, 0)),
            pl.BlockSpec((16, 16), _full16),
            pl.BlockSpec((16, 16), _full16),
            pl.BlockSpec((_M * _C, _O), _full16),
            pl.BlockSpec((1, 16), _full16),
            pl.BlockSpec((1, 16), _full16),
            pl.BlockSpec((1, 16), _full16),
            pl.BlockSpec((1, 16), _full16),
        ],
        out_specs=pl.BlockSpec((_TS3, _O), lambda i: (i, 0)),
        out_shape=jax.ShapeDtypeStruct((_S, _O), F32),
        interpret=interpret,
    )


def _prep_weights(W_fc1, W_fc2, W_fc3, W_cv):
    w1r = jnp.pad(W_fc1, ((0, 0), (0, 13))).T
    w2a = W_fc2[:, :16].T
    w2b = W_fc2[:, 16:].T
    w3a = W_fc3[:, :16].T
    w3b = W_fc3[:, 16:].T
    wcv = jnp.transpose(W_cv, (2, 1, 0)).reshape(_M * _C, _O)
    return w1r, w2a, w2b, w3a, w3b, wcv


def kernel(x, pos, support_points, neighbors_indices, W_fc1, W_fc2, W_fc3,
           g1, b1, g2, b2, alpha, beta, norm_radius, W_cv):
    xt = x[0].T
    pt = jnp.pad(pos[0].T, ((0, 0), (0, 13)))
    spt = jnp.pad(support_points[0].T, ((0, 0), (0, 13)))
    idx = neighbors_indices[0].astype(jnp.int32).reshape(_SK)

    pg = _sc_gather_build(16)(pt, idx)

    w1r, w2a, w2b, w3a, w3b, wcv = _prep_weights(W_fc1, W_fc2, W_fc3, W_cv)
    scl = jnp.concatenate([alpha, beta, norm_radius,
                           jnp.zeros((1,), F32)]).reshape(1, 4)
    g1r, b1r = g1.reshape(1, 16), b1.reshape(1, 16)
    g2r, b2r = g2.reshape(1, 16), b2.reshape(1, 16)

    s1, q1 = _make_pass1()(pg, spt, w1r, scl)
    xg = _sc_gather_build(_C)(xt, idx)
    s2, q2, mz = _make_pass2()(pg, spt, w1r, w2a, w2b, scl, g1r, b1r, s1, q1)
    out2d = _make_pass3()(mz, xg, w3a, w3b, wcv, g2r, b2r, s2, q2)
    return out2d.T[None, :, :]
